# Initial kernel scaffold; baseline (speedup 1.0000x reference)
import jax
import jax.numpy as jnp
from jax.experimental import pallas as pl


def kernel(inputs, weight):
    raise NotImplementedError("write your pallas kernel here")


# speedup vs baseline: 1.1376x; 1.1376x over previous
import jax
import jax.numpy as jnp
from jax.experimental import pallas as pl
from jax.experimental.pallas import tpu as pltpu
from jax.experimental.pallas import tpu_sc as plsc

NUM_K = 8192
DIM = 64
BATCH = 8
TOKENS_PER_ROW = 1024
N_TOKENS = BATCH * TOKENS_PER_ROW
KBLK = 1024
TBLK = 1024


def _argmin_body(u_ref, wt_ref, a_ref, b_ref, min_ref, arg_ref):
    k = pl.program_id(1)
    c = jax.lax.dot_general(
        u_ref[...], wt_ref[...], (((1,), (0,)), ((), ())),
        preferred_element_type=jnp.float32)
    dist = (a_ref[...] + b_ref[...]) - 2.0 * c
    bmin = jnp.min(dist, axis=1, keepdims=True)
    iota = jax.lax.broadcasted_iota(jnp.int32, dist.shape, 1)
    barg = jnp.min(jnp.where(dist == bmin, iota, jnp.int32(2 ** 30)),
                   axis=1, keepdims=True) + k * KBLK

    @pl.when(k == 0)
    def _():
        min_ref[...] = bmin
        arg_ref[...] = barg

    @pl.when(k != 0)
    def _():
        better = bmin < min_ref[...]
        arg_ref[...] = jnp.where(better, barg, arg_ref[...])
        min_ref[...] = jnp.where(better, bmin, min_ref[...])


def _sort_body(key_ref, enc_ref, out_ref):
    keys = key_ref[...]
    vals = enc_ref[...]
    n = keys.shape[1]
    pos = jax.lax.broadcasted_iota(jnp.int32, keys.shape, 1)
    i = pos

    def partner(x, j, lower):
        fwd = jnp.roll(x, -j, axis=1)
        bwd = jnp.roll(x, j, axis=1)
        return jnp.where(lower, fwd, bwd)

    k = 2
    while k <= n:
        j = k // 2
        while j >= 1:
            lower = (i & j) == 0
            pk = partner(keys, j, lower)
            pp = partner(pos, j, lower)
            pv = partner(vals, j, lower)
            asc = (i & k) == 0
            lt = (pk < keys) | ((pk == keys) & (pp < pos))
            take = lt == (lower == asc)
            keys = jnp.where(take, pk, keys)
            pos = jnp.where(take, pp, pos)
            vals = jnp.where(take, pv, vals)
            j //= 2
        k *= 2
    out_ref[...] = vals


def _sc_gather(weight, idx2d, n_rows):
    mesh = plsc.VectorSubcoreMesh(core_axis_name="core",
                                  subcore_axis_name="subcore")
    window = 128
    wpad = jnp.pad(weight, ((0, 0), (0, 128 - DIM)))

    @pl.kernel(out_type=jax.ShapeDtypeStruct((n_rows, 128), weight.dtype),
               mesh=mesh)
    def kern(x_hbm, i_hbm, o_hbm):
        def body(i_vmem, o_vmem):
            pltpu.sync_copy(x_hbm.at[i_vmem.at[0]], o_vmem)

        pltpu.emit_pipeline(
            body,
            grid=(n_rows // window,),
            in_specs=[pl.BlockSpec((1, window), lambda i: (0, i))],
            out_specs=[pl.BlockSpec((window, 128), lambda i: (i, 0))],
            core_axis_name=("core", "subcore"),
            dimension_semantics=(pltpu.PARALLEL,),
        )(i_hbm, o_hbm)

    return kern(wpad, idx2d)[:, :DIM]


def kernel(inputs, weight):
    input_shape = inputs.shape
    flat = inputs.reshape(-1, DIM)
    wt = weight.T
    a = jnp.sum(flat ** 2, axis=1, keepdims=True)
    b = jnp.sum(weight ** 2, axis=1)[None, :]

    grid = (N_TOKENS // TBLK, NUM_K // KBLK)
    mins, args = pl.pallas_call(
        _argmin_body,
        grid=grid,
        in_specs=[
            pl.BlockSpec((TBLK, DIM), lambda t, k: (t, 0)),
            pl.BlockSpec((DIM, KBLK), lambda t, k: (0, k)),
            pl.BlockSpec((TBLK, 1), lambda t, k: (t, 0)),
            pl.BlockSpec((1, KBLK), lambda t, k: (0, k)),
        ],
        out_specs=[
            pl.BlockSpec((TBLK, 1), lambda t, k: (t, 0)),
            pl.BlockSpec((TBLK, 1), lambda t, k: (t, 0)),
        ],
        out_shape=[
            jax.ShapeDtypeStruct((N_TOKENS, 1), jnp.float32),
            jax.ShapeDtypeStruct((N_TOKENS, 1), jnp.int32),
        ],
    )(flat, wt, a, b)

    keys = mins[:, 0].reshape(BATCH, TOKENS_PER_ROW)
    enc0 = jnp.broadcast_to(args[:TOKENS_PER_ROW, 0][None, :],
                            (BATCH, TOKENS_PER_ROW))

    final_idx = pl.pallas_call(
        _sort_body,
        in_specs=[
            pl.BlockSpec((BATCH, TOKENS_PER_ROW), lambda: (0, 0)),
            pl.BlockSpec((BATCH, TOKENS_PER_ROW), lambda: (0, 0)),
        ],
        out_specs=pl.BlockSpec((BATCH, TOKENS_PER_ROW), lambda: (0, 0)),
        out_shape=jax.ShapeDtypeStruct((BATCH, TOKENS_PER_ROW), jnp.int32),
    )(keys, enc0)

    gathered = _sc_gather(weight, final_idx.reshape(1, N_TOKENS), N_TOKENS)
    return gathered.reshape(input_shape)


# speedup vs baseline: 1.5345x; 1.3489x over previous
import jax
import jax.numpy as jnp
from jax.experimental import pallas as pl
from jax.experimental.pallas import tpu as pltpu
from jax.experimental.pallas import tpu_sc as plsc

NUM_K = 8192
DIM = 64
BATCH = 8
TOKENS_PER_ROW = 1024
N_TOKENS = BATCH * TOKENS_PER_ROW
KBLK = 1024
TBLK = 1024


def _argmin_body(u_ref, wt_ref, a_ref, b_ref, min_ref, arg_ref):
    t = pl.program_id(0)
    k = pl.program_id(1)
    c = jax.lax.dot_general(
        u_ref[...], wt_ref[...], (((1,), (0,)), ((), ())),
        preferred_element_type=jnp.float32)
    dist = (a_ref[...] + b_ref[...]) - 2.0 * c
    bmin = jnp.min(dist, axis=1, keepdims=True)

    @pl.when(t == 0)
    def _():
        iota = jax.lax.broadcasted_iota(jnp.int32, dist.shape, 1)
        barg = jnp.min(jnp.where(dist == bmin, iota, jnp.int32(2 ** 30)),
                       axis=1, keepdims=True) + k * KBLK

        @pl.when(k == 0)
        def _():
            arg_ref[...] = barg

        @pl.when(k != 0)
        def _():
            arg_ref[...] = jnp.where(bmin < min_ref[...], barg, arg_ref[...])

    @pl.when(k == 0)
    def _():
        min_ref[...] = bmin

    @pl.when(k != 0)
    def _():
        min_ref[...] = jnp.minimum(min_ref[...], bmin)


def _sort_body(key_ref, enc_ref, out_ref):
    keys = key_ref[...]
    vals = enc_ref[...]
    n = keys.shape[1]
    pos = jax.lax.broadcasted_iota(jnp.int32, keys.shape, 1)
    i = pos

    def partner(x, j, lower):
        fwd = jnp.roll(x, -j, axis=1)
        bwd = jnp.roll(x, j, axis=1)
        return jnp.where(lower, fwd, bwd)

    k = 2
    while k <= n:
        j = k // 2
        while j >= 1:
            lower = (i & j) == 0
            pk = partner(keys, j, lower)
            pp = partner(pos, j, lower)
            pv = partner(vals, j, lower)
            asc = (i & k) == 0
            lt = (pk < keys) | ((pk == keys) & (pp < pos))
            take = lt == (lower == asc)
            keys = jnp.where(take, pk, keys)
            pos = jnp.where(take, pp, pos)
            vals = jnp.where(take, pv, vals)
            j //= 2
        k *= 2
    out_ref[...] = vals


def _sc_gather(weight, idx2d, n_rows):
    mesh = plsc.VectorSubcoreMesh(core_axis_name="core",
                                  subcore_axis_name="subcore")
    window = 128
    wpad = jnp.pad(weight, ((0, 0), (0, 128 - DIM)))

    @pl.kernel(out_type=jax.ShapeDtypeStruct((n_rows, 128), weight.dtype),
               mesh=mesh)
    def kern(x_hbm, i_hbm, o_hbm):
        def body(i_vmem, o_vmem):
            pltpu.sync_copy(x_hbm.at[i_vmem.at[0]], o_vmem)

        pltpu.emit_pipeline(
            body,
            grid=(n_rows // window,),
            in_specs=[pl.BlockSpec((1, window), lambda i: (0, i))],
            out_specs=[pl.BlockSpec((window, 128), lambda i: (i, 0))],
            core_axis_name=("core", "subcore"),
            dimension_semantics=(pltpu.PARALLEL,),
        )(i_hbm, o_hbm)

    return kern(wpad, idx2d)[:, :DIM]


def kernel(inputs, weight):
    input_shape = inputs.shape
    flat = inputs.reshape(-1, DIM)
    wt = weight.T
    a = jnp.sum(flat ** 2, axis=1, keepdims=True)
    b = jnp.sum(weight ** 2, axis=1)[None, :]

    grid = (N_TOKENS // TBLK, NUM_K // KBLK)
    mins, args = pl.pallas_call(
        _argmin_body,
        grid=grid,
        in_specs=[
            pl.BlockSpec((TBLK, DIM), lambda t, k: (t, 0)),
            pl.BlockSpec((DIM, KBLK), lambda t, k: (0, k)),
            pl.BlockSpec((TBLK, 1), lambda t, k: (t, 0)),
            pl.BlockSpec((1, KBLK), lambda t, k: (0, k)),
        ],
        out_specs=[
            pl.BlockSpec((TBLK, 1), lambda t, k: (t, 0)),
            pl.BlockSpec((TBLK, 1), lambda t, k: (0, 0)),
        ],
        out_shape=[
            jax.ShapeDtypeStruct((N_TOKENS, 1), jnp.float32),
            jax.ShapeDtypeStruct((TBLK, 1), jnp.int32),
        ],
    )(flat, wt, a, b)

    keys = mins[:, 0].reshape(BATCH, TOKENS_PER_ROW)
    enc0 = jnp.broadcast_to(args[:TOKENS_PER_ROW, 0][None, :],
                            (BATCH, TOKENS_PER_ROW))

    final_idx = pl.pallas_call(
        _sort_body,
        in_specs=[
            pl.BlockSpec((BATCH, TOKENS_PER_ROW), lambda: (0, 0)),
            pl.BlockSpec((BATCH, TOKENS_PER_ROW), lambda: (0, 0)),
        ],
        out_specs=pl.BlockSpec((BATCH, TOKENS_PER_ROW), lambda: (0, 0)),
        out_shape=jax.ShapeDtypeStruct((BATCH, TOKENS_PER_ROW), jnp.int32),
    )(keys, enc0)

    gathered = _sc_gather(weight, final_idx.reshape(1, N_TOKENS), N_TOKENS)
    return gathered.reshape(input_shape)


# speedup vs baseline: 1.5767x; 1.0275x over previous
import jax
import jax.numpy as jnp
from jax.experimental import pallas as pl
from jax.experimental.pallas import tpu as pltpu
from jax.experimental.pallas import tpu_sc as plsc

NUM_K = 8192
DIM = 64
BATCH = 8
TOKENS_PER_ROW = 1024
N_TOKENS = BATCH * TOKENS_PER_ROW
KBLK = 1024
TBLK = 1024


def _argmin_body(u_ref, wt2_ref, a_ref, b_ref, min_ref, arg_ref):
    t = pl.program_id(0)
    k = pl.program_id(1)
    c2 = jax.lax.dot_general(
        u_ref[...], wt2_ref[...], (((1,), (0,)), ((), ())),
        preferred_element_type=jnp.float32)
    dist = (a_ref[...] + b_ref[...]) - c2
    bmin = jnp.min(dist, axis=1, keepdims=True)

    @pl.when(t == 0)
    def _():
        iota = jax.lax.broadcasted_iota(
            jnp.int32, dist.shape, 1).astype(jnp.float32)
        barg_f = jnp.min(jnp.where(dist == bmin, iota, jnp.float32(2.0 ** 30)),
                         axis=1, keepdims=True)
        barg = barg_f.astype(jnp.int32) + k * KBLK

        @pl.when(k == 0)
        def _():
            arg_ref[...] = barg

        @pl.when(k != 0)
        def _():
            arg_ref[...] = jnp.where(bmin < min_ref[...], barg, arg_ref[...])

    @pl.when(k == 0)
    def _():
        min_ref[...] = bmin

    @pl.when(k != 0)
    def _():
        min_ref[...] = jnp.minimum(min_ref[...], bmin)


def _sort_body(key_ref, enc_ref, out_ref):
    keys = key_ref[...]
    vals = enc_ref[...]
    n = keys.shape[1]
    pos = jax.lax.broadcasted_iota(jnp.int32, keys.shape, 1)
    i = pos

    def partner(x, j, lower):
        fwd = jnp.roll(x, -j, axis=1)
        bwd = jnp.roll(x, j, axis=1)
        return jnp.where(lower, fwd, bwd)

    k = 2
    while k <= n:
        j = k // 2
        while j >= 1:
            lower = (i & j) == 0
            pk = partner(keys, j, lower)
            pp = partner(pos, j, lower)
            pv = partner(vals, j, lower)
            asc = (i & k) == 0
            lt = (pk < keys) | ((pk == keys) & (pp < pos))
            take = lt == (lower == asc)
            keys = jnp.where(take, pk, keys)
            pos = jnp.where(take, pp, pos)
            vals = jnp.where(take, pv, vals)
            j //= 2
        k *= 2
    out_ref[...] = vals


def _sc_gather(weight, idx2d, n_rows):
    mesh = plsc.VectorSubcoreMesh(core_axis_name="core",
                                  subcore_axis_name="subcore")
    window = 128
    wpad = jnp.pad(weight, ((0, 0), (0, 128 - DIM)))

    @pl.kernel(out_type=jax.ShapeDtypeStruct((n_rows, 128), weight.dtype),
               mesh=mesh)
    def kern(x_hbm, i_hbm, o_hbm):
        def body(i_vmem, o_vmem):
            pltpu.sync_copy(x_hbm.at[i_vmem.at[0]], o_vmem)

        pltpu.emit_pipeline(
            body,
            grid=(n_rows // window,),
            in_specs=[pl.BlockSpec((1, window), lambda i: (0, i))],
            out_specs=[pl.BlockSpec((window, 128), lambda i: (i, 0))],
            core_axis_name=("core", "subcore"),
            dimension_semantics=(pltpu.PARALLEL,),
        )(i_hbm, o_hbm)

    return kern(wpad, idx2d)[:, :DIM]


def kernel(inputs, weight):
    input_shape = inputs.shape
    flat = inputs.reshape(-1, DIM)
    wt2 = 2.0 * weight.T
    a = jnp.sum(flat ** 2, axis=1, keepdims=True)
    b = jnp.sum(weight ** 2, axis=1)[None, :]

    grid = (N_TOKENS // TBLK, NUM_K // KBLK)
    mins, args = pl.pallas_call(
        _argmin_body,
        grid=grid,
        in_specs=[
            pl.BlockSpec((TBLK, DIM), lambda t, k: (t, 0)),
            pl.BlockSpec((DIM, KBLK), lambda t, k: (0, k)),
            pl.BlockSpec((TBLK, 1), lambda t, k: (t, 0)),
            pl.BlockSpec((1, KBLK), lambda t, k: (0, k)),
        ],
        out_specs=[
            pl.BlockSpec((TBLK, 1), lambda t, k: (t, 0)),
            pl.BlockSpec((TBLK, 1), lambda t, k: (0, 0)),
        ],
        out_shape=[
            jax.ShapeDtypeStruct((N_TOKENS, 1), jnp.float32),
            jax.ShapeDtypeStruct((TBLK, 1), jnp.int32),
        ],
        compiler_params=pltpu.CompilerParams(
            dimension_semantics=("parallel", "arbitrary")),
    )(flat, wt2, a, b)

    keys = mins[:, 0].reshape(BATCH, TOKENS_PER_ROW)
    enc0 = jnp.broadcast_to(args[:TOKENS_PER_ROW, 0][None, :],
                            (BATCH, TOKENS_PER_ROW))

    final_idx = pl.pallas_call(
        _sort_body,
        in_specs=[
            pl.BlockSpec((BATCH, TOKENS_PER_ROW), lambda: (0, 0)),
            pl.BlockSpec((BATCH, TOKENS_PER_ROW), lambda: (0, 0)),
        ],
        out_specs=pl.BlockSpec((BATCH, TOKENS_PER_ROW), lambda: (0, 0)),
        out_shape=jax.ShapeDtypeStruct((BATCH, TOKENS_PER_ROW), jnp.int32),
    )(keys, enc0)

    gathered = _sc_gather(weight, final_idx.reshape(1, N_TOKENS), N_TOKENS)
    return gathered.reshape(input_shape)
